# Initial kernel scaffold; baseline (speedup 1.0000x reference)
import jax
import jax.numpy as jnp
from jax.experimental import pallas as pl


def kernel(input_ids, attention_mask, emb_table, W, b):
    raise NotImplementedError("write your pallas kernel here")


# speedup vs baseline: 16.0562x; 16.0562x over previous
import functools

import jax
import jax.numpy as jnp
from jax import lax
from jax.experimental import pallas as pl
from jax.experimental.pallas import tpu as pltpu
from jax.experimental.pallas import tpu_sc as plsc

VOCAB = 30522
EMBED_DIM = 128
SEQ_LEN = 512
BATCH = 1024
FC_OUT = 64

NUM_CORES = 2
NUM_SUBCORES = 16
NW = NUM_CORES * NUM_SUBCORES
BPW = BATCH // NW
CHUNK = 128
CPB = SEQ_LEN // CHUNK
NCH = BPW * CPB
LANES = 16
NVR = FC_OUT // LANES

ROW_BLK = 1024


def _project_kernel(t_ref, w_ref, p_ref):
    p_ref[...] = jnp.dot(t_ref[...], w_ref[...],
                         preferred_element_type=jnp.float32)


def _project(table, w):
    grid = (pl.cdiv(VOCAB, ROW_BLK),)
    return pl.pallas_call(
        _project_kernel,
        grid=grid,
        in_specs=[
            pl.BlockSpec((ROW_BLK, EMBED_DIM), lambda i: (i, 0)),
            pl.BlockSpec((EMBED_DIM, FC_OUT), lambda i: (0, 0)),
        ],
        out_specs=pl.BlockSpec((ROW_BLK, FC_OUT), lambda i: (i, 0)),
        out_shape=jax.ShapeDtypeStruct((VOCAB, FC_OUT), jnp.float32),
    )(table, w)


_MESH = plsc.VectorSubcoreMesh(core_axis_name="c", subcore_axis_name="s",
                               num_cores=NUM_CORES, num_subcores=NUM_SUBCORES)


@functools.partial(
    pl.kernel,
    out_type=jax.ShapeDtypeStruct((BATCH, FC_OUT), jnp.float32),
    mesh=_MESH,
    scratch_types=[
        pltpu.VMEM((NCH, CHUNK), jnp.int32),
        pltpu.VMEM((CPB, CHUNK, FC_OUT), jnp.float32),
        pltpu.VMEM((BPW, FC_OUT), jnp.float32),
        pltpu.VMEM((FC_OUT,), jnp.float32),
        pltpu.SemaphoreType.DMA,
        pltpu.SemaphoreType.DMA,
        pltpu.SemaphoreType.DMA,
        pltpu.SemaphoreType.DMA,
    ],
    compiler_params=pltpu.CompilerParams(use_tc_tiling_on_sc=False),
)
def _gather_mean(ids_hbm, p_hbm, b_hbm, out_hbm,
                 idx_v, rows_v, out_v, bias_v, s0, s1, s2, s3):
    sems = (s0, s1, s2, s3)
    wid = lax.axis_index("s") * NUM_CORES + lax.axis_index("c")

    pltpu.sync_copy(b_hbm, bias_v)
    pltpu.sync_copy(ids_hbm.at[pl.ds(wid * NCH, NCH)], idx_v)

    def _copy(j, c):
        return pltpu.make_async_copy(
            p_hbm.at[idx_v.at[j]], rows_v.at[c], sems[c])

    for c in range(CPB):
        _copy(c, c).start()

    inv = jnp.float32(1.0 / SEQ_LEN)

    def batch_body(bi, _):
        accs = [jnp.zeros((LANES,), jnp.float32) for _ in range(NVR)]
        for c in range(CPB):
            j = bi * CPB + c
            _copy(j, c).wait()

            def row_body(r, a):
                return tuple(
                    a[k] + rows_v[c, r, pl.ds(k * LANES, LANES)]
                    for k in range(NVR))

            accs = list(lax.fori_loop(0, CHUNK, row_body, tuple(accs),
                                      unroll=8))

            @pl.when(bi + 1 < BPW)
            def _():
                _copy(j + CPB, c).start()

        for k in range(NVR):
            v = accs[k] * inv + bias_v[pl.ds(k * LANES, LANES)]
            out_v[bi, pl.ds(k * LANES, LANES)] = jnp.maximum(v, 0.0)
        return 0

    lax.fori_loop(0, BPW, batch_body, 0)
    pltpu.sync_copy(out_v, out_hbm.at[pl.ds(wid * BPW, BPW)])


def kernel(input_ids, attention_mask, emb_table, W, b):
    del attention_mask
    p = _project(emb_table, W)
    ids = input_ids.astype(jnp.int32).reshape(-1, CHUNK)
    return _gather_mean(ids, p, b)


# speedup vs baseline: 16.6103x; 1.0345x over previous
import functools

import jax
import jax.numpy as jnp
import numpy as np
from jax import lax
from jax.experimental import pallas as pl
from jax.experimental.pallas import tpu as pltpu
from jax.experimental.pallas import tpu_sc as plsc

VOCAB = 30522
EMBED_DIM = 128
SEQ_LEN = 512
BATCH = 1024
FC_OUT = 64

NUM_CORES = 2
NUM_SUBCORES = 16
NW = NUM_CORES * NUM_SUBCORES
BPW = BATCH // NW
CHUNK = 128
CPB = SEQ_LEN // CHUNK
NCH = BPW * CPB
LANES = 16
NVR = FC_OUT // LANES

ROW_BLK = 1024

_PERM = np.empty((FC_OUT,), dtype=np.int32)
for _h in range(2):
    for _i in range(16):
        _PERM[32 * _h + 2 * _i] = 32 * _h + _i
        _PERM[32 * _h + 2 * _i + 1] = 32 * _h + 16 + _i


def _project_kernel(t_ref, w_ref, p_ref):
    p_ref[...] = jnp.dot(t_ref[...], w_ref[...],
                         preferred_element_type=jnp.float32
                         ).astype(jnp.bfloat16)


def _project(table, w):
    grid = (pl.cdiv(VOCAB, ROW_BLK),)
    return pl.pallas_call(
        _project_kernel,
        grid=grid,
        in_specs=[
            pl.BlockSpec((ROW_BLK, EMBED_DIM), lambda i: (i, 0)),
            pl.BlockSpec((EMBED_DIM, FC_OUT), lambda i: (0, 0)),
        ],
        out_specs=pl.BlockSpec((ROW_BLK, FC_OUT), lambda i: (i, 0)),
        out_shape=jax.ShapeDtypeStruct((VOCAB, FC_OUT), jnp.bfloat16),
    )(table, w)


_MESH = plsc.VectorSubcoreMesh(core_axis_name="c", subcore_axis_name="s",
                               num_cores=NUM_CORES, num_subcores=NUM_SUBCORES)


@functools.partial(
    pl.kernel,
    out_type=jax.ShapeDtypeStruct((BATCH, FC_OUT), jnp.float32),
    mesh=_MESH,
    scratch_types=[
        pltpu.VMEM((NCH, CHUNK), jnp.int32),
        pltpu.VMEM((CPB, CHUNK, FC_OUT), jnp.bfloat16),
        pltpu.VMEM((BPW, FC_OUT), jnp.float32),
        pltpu.VMEM((FC_OUT,), jnp.float32),
        pltpu.SemaphoreType.DMA,
        pltpu.SemaphoreType.DMA,
        pltpu.SemaphoreType.DMA,
        pltpu.SemaphoreType.DMA,
    ],
    compiler_params=pltpu.CompilerParams(use_tc_tiling_on_sc=False,
                                         needs_layout_passes=False),
)
def _gather_mean(ids_hbm, p_hbm, b_hbm, out_hbm,
                 idx_v, rows_v, out_v, bias_v, s0, s1, s2, s3):
    sems = (s0, s1, s2, s3)
    wid = lax.axis_index("s") * NUM_CORES + lax.axis_index("c")

    pltpu.sync_copy(b_hbm, bias_v)
    pltpu.sync_copy(ids_hbm.at[pl.ds(wid * NCH, NCH)], idx_v)

    def _copy(j, c):
        return pltpu.make_async_copy(
            p_hbm.at[idx_v.at[j]], rows_v.at[c], sems[c])

    for c in range(CPB):
        _copy(c, c).start()

    inv = jnp.float32(1.0 / SEQ_LEN)

    def batch_body(bi, _):
        accs = [jnp.zeros((LANES,), jnp.float32) for _ in range(NVR)]
        for c in range(CPB):
            j = bi * CPB + c
            _copy(j, c).wait()

            def row_body(r, a):
                out = list(a)
                for h in range(2):
                    pair = rows_v[c, r, pl.ds(32 * h, 32)]
                    e0, e1 = plsc.unpack(pair,
                                         format=plsc.PackFormat.INTERLEAVED)
                    out[2 * h] = out[2 * h] + e0
                    out[2 * h + 1] = out[2 * h + 1] + e1
                return tuple(out)

            accs = list(lax.fori_loop(0, CHUNK, row_body, tuple(accs),
                                      unroll=8))

            @pl.when(bi + 1 < BPW)
            def _():
                _copy(j + CPB, c).start()

        for k in range(NVR):
            v = accs[k] * inv + bias_v[pl.ds(k * LANES, LANES)]
            out_v[bi, pl.ds(k * LANES, LANES)] = jnp.maximum(v, 0.0)
        return 0

    lax.fori_loop(0, BPW, batch_body, 0)
    pltpu.sync_copy(out_v, out_hbm.at[pl.ds(wid * BPW, BPW)])


def kernel(input_ids, attention_mask, emb_table, W, b):
    del attention_mask
    p = _project(emb_table, W[:, _PERM])
    ids = input_ids.astype(jnp.int32).reshape(-1, CHUNK)
    return _gather_mean(ids, p, b)


# speedup vs baseline: 16.6172x; 1.0004x over previous
import functools

import jax
import jax.numpy as jnp
from jax import lax
from jax.experimental import pallas as pl
from jax.experimental.pallas import tpu as pltpu
from jax.experimental.pallas import tpu_sc as plsc

VOCAB = 30522
EMBED_DIM = 128
SEQ_LEN = 512
BATCH = 1024
FC_OUT = 64

NUM_CORES = 2
NUM_SUBCORES = 16
NW = NUM_CORES * NUM_SUBCORES
BPW = BATCH // NW
CHUNK = 128
CPB = SEQ_LEN // CHUNK
NCH = BPW * CPB
LANES = 16
NVR = FC_OUT // LANES

ROW_BLK = 1024


def _project_kernel(t_ref, w_ref, p_ref):
    p_ref[...] = jnp.dot(t_ref[...], w_ref[...],
                         preferred_element_type=jnp.float32
                         ).astype(jnp.bfloat16)


def _project(table, w):
    grid = (pl.cdiv(VOCAB, ROW_BLK),)
    return pl.pallas_call(
        _project_kernel,
        grid=grid,
        in_specs=[
            pl.BlockSpec((ROW_BLK, EMBED_DIM), lambda i: (i, 0)),
            pl.BlockSpec((EMBED_DIM, FC_OUT), lambda i: (0, 0)),
        ],
        out_specs=pl.BlockSpec((ROW_BLK, FC_OUT), lambda i: (i, 0)),
        out_shape=jax.ShapeDtypeStruct((VOCAB, FC_OUT), jnp.bfloat16),
    )(table, w)


_MESH = plsc.VectorSubcoreMesh(core_axis_name="c", subcore_axis_name="s",
                               num_cores=NUM_CORES, num_subcores=NUM_SUBCORES)


@functools.partial(
    pl.kernel,
    out_type=jax.ShapeDtypeStruct((BATCH, FC_OUT), jnp.float32),
    mesh=_MESH,
    scratch_types=[
        pltpu.VMEM((BPW, SEQ_LEN), jnp.int32),
        pltpu.VMEM((CPB, CHUNK, FC_OUT), jnp.bfloat16),
        pltpu.VMEM((BPW, FC_OUT), jnp.float32),
        pltpu.VMEM((FC_OUT,), jnp.float32),
        pltpu.SemaphoreType.DMA,
        pltpu.SemaphoreType.DMA,
        pltpu.SemaphoreType.DMA,
        pltpu.SemaphoreType.DMA,
    ],
    compiler_params=pltpu.CompilerParams(use_tc_tiling_on_sc=False,
                                         needs_layout_passes=False),
)
def _gather_mean(ids_hbm, p_hbm, b_hbm, out_hbm,
                 idx_v, rows_v, out_v, bias_v, s0, s1, s2, s3):
    sems = (s0, s1, s2, s3)
    wid = lax.axis_index("s") * NUM_CORES + lax.axis_index("c")
    pltpu.sync_copy(b_hbm, bias_v)
    pltpu.sync_copy(ids_hbm.at[pl.ds(wid * BPW, BPW)], idx_v)

    def _copy(bi, c):
        return pltpu.make_async_copy(
            p_hbm.at[idx_v.at[bi, pl.ds(c * CHUNK, CHUNK)]],
            rows_v.at[c], sems[c])

    for c in range(CPB):
        _copy(0, c).start()

    inv = jnp.float32(1.0 / SEQ_LEN)
    iota = lax.iota(jnp.int32, LANES)
    cols = [[32 * h + 2 * iota + g for g in range(2)] for h in range(2)]
    biases = [[plsc.load_gather(bias_v, [cols[h][g]]) for g in range(2)]
              for h in range(2)]

    def batch_body(bi, _):
        accs = [jnp.zeros((LANES,), jnp.float32) for _ in range(NVR)]
        for c in range(CPB):
            _copy(bi, c).wait()

            def row_body(r, a):
                out = list(a)
                for h in range(2):
                    pair = rows_v[c, r, pl.ds(32 * h, 32)]
                    e0, e1 = plsc.unpack(pair,
                                         format=plsc.PackFormat.INTERLEAVED)
                    out[2 * h] = out[2 * h] + e0
                    out[2 * h + 1] = out[2 * h + 1] + e1
                return tuple(out)

            accs = list(lax.fori_loop(0, CHUNK, row_body, tuple(accs),
                                      unroll=8))

            @pl.when(bi + 1 < BPW)
            def _():
                _copy(bi + 1, c).start()

        rowi = jnp.full((LANES,), 0, jnp.int32) + bi
        for h in range(2):
            for g in range(2):
                v = accs[2 * h + g] * inv + biases[h][g]
                plsc.store_scatter(out_v, [rowi, cols[h][g]],
                                   jnp.maximum(v, 0.0))
        return 0

    lax.fori_loop(0, BPW, batch_body, 0)
    pltpu.sync_copy(out_v, out_hbm.at[pl.ds(wid * BPW, BPW)])


def kernel(input_ids, attention_mask, emb_table, W, b):
    del attention_mask
    p = _project(emb_table, W)
    return _gather_mean(input_ids.astype(jnp.int32), p, b)


# speedup vs baseline: 16.8432x; 1.0136x over previous
import functools

import jax
import jax.numpy as jnp
from jax import lax
from jax.experimental import pallas as pl
from jax.experimental.pallas import tpu as pltpu
from jax.experimental.pallas import tpu_sc as plsc

VOCAB = 30522
EMBED_DIM = 128
SEQ_LEN = 512
BATCH = 1024
FC_OUT = 64

NUM_CORES = 2
NUM_SUBCORES = 16
NW = NUM_CORES * NUM_SUBCORES
BPW = BATCH // NW
CHUNK = 128
CPB = SEQ_LEN // CHUNK
NCH = BPW * CPB
LANES = 16
NVR = FC_OUT // LANES

ROW_BLK = 1024


def _project_kernel(t_ref, w_ref, p_ref):
    p_ref[...] = jnp.dot(t_ref[...], w_ref[...],
                         preferred_element_type=jnp.float32
                         ).astype(jnp.bfloat16)


def _project(table, w):
    grid = (pl.cdiv(VOCAB, ROW_BLK),)
    return pl.pallas_call(
        _project_kernel,
        grid=grid,
        in_specs=[
            pl.BlockSpec((ROW_BLK, EMBED_DIM), lambda i: (i, 0)),
            pl.BlockSpec((EMBED_DIM, FC_OUT), lambda i: (0, 0)),
        ],
        out_specs=pl.BlockSpec((ROW_BLK, FC_OUT), lambda i: (i, 0)),
        out_shape=jax.ShapeDtypeStruct((VOCAB, FC_OUT), jnp.bfloat16),
    )(table, w)


_MESH = plsc.VectorSubcoreMesh(core_axis_name="c", subcore_axis_name="s",
                               num_cores=NUM_CORES, num_subcores=NUM_SUBCORES)


@functools.partial(
    pl.kernel,
    out_type=jax.ShapeDtypeStruct((BATCH, FC_OUT), jnp.float32),
    mesh=_MESH,
    scratch_types=[
        pltpu.VMEM((BPW, SEQ_LEN), jnp.int32),
        pltpu.VMEM((CPB, CHUNK, FC_OUT), jnp.bfloat16),
        pltpu.VMEM((BPW, FC_OUT), jnp.float32),
        pltpu.VMEM((FC_OUT,), jnp.float32),
        pltpu.SemaphoreType.DMA,
        pltpu.SemaphoreType.DMA,
        pltpu.SemaphoreType.DMA,
        pltpu.SemaphoreType.DMA,
    ],
    compiler_params=pltpu.CompilerParams(use_tc_tiling_on_sc=False,
                                         needs_layout_passes=False),
)
def _gather_mean(ids_hbm, p_hbm, b_hbm, out_hbm,
                 idx_v, rows_v, out_v, bias_v, s0, s1, s2, s3):
    sems = (s0, s1, s2, s3)
    wid = lax.axis_index("s") * NUM_CORES + lax.axis_index("c")
    pltpu.sync_copy(b_hbm, bias_v)
    pltpu.sync_copy(ids_hbm.at[pl.ds(wid * BPW, BPW)], idx_v)

    def _copy(bi, c):
        return pltpu.make_async_copy(
            p_hbm.at[idx_v.at[bi, pl.ds(c * CHUNK, CHUNK)]],
            rows_v.at[c], sems[c])

    for c in range(CPB):
        _copy(0, c).start()

    inv = jnp.float32(1.0 / SEQ_LEN)
    iota = lax.iota(jnp.int32, LANES)
    cols = [[32 * h + 2 * iota + g for g in range(2)] for h in range(2)]
    biases = [[plsc.load_gather(bias_v, [cols[h][g]]) for g in range(2)]
              for h in range(2)]

    def batch_body(bi, _):
        accs = [jnp.zeros((LANES,), jnp.float32) for _ in range(NVR)]
        for c in range(CPB):
            _copy(bi, c).wait()

            def row_body(r4, a):
                out = list(a)
                r = r4 * 4
                for h in range(2):
                    s = rows_v[c, r, pl.ds(32 * h, 32)]
                    for d in range(1, 4):
                        s = s + rows_v[c, r + d, pl.ds(32 * h, 32)]
                    e0, e1 = plsc.unpack(s,
                                         format=plsc.PackFormat.INTERLEAVED)
                    out[2 * h] = out[2 * h] + e0
                    out[2 * h + 1] = out[2 * h + 1] + e1
                return tuple(out)

            accs = list(lax.fori_loop(0, CHUNK // 4, row_body, tuple(accs),
                                      unroll=4))

            @pl.when(bi + 1 < BPW)
            def _():
                _copy(bi + 1, c).start()

        rowi = jnp.full((LANES,), 0, jnp.int32) + bi
        for h in range(2):
            for g in range(2):
                v = accs[2 * h + g] * inv + biases[h][g]
                plsc.store_scatter(out_v, [rowi, cols[h][g]],
                                   jnp.maximum(v, 0.0))
        return 0

    lax.fori_loop(0, BPW, batch_body, 0)
    pltpu.sync_copy(out_v, out_hbm.at[pl.ds(wid * BPW, BPW)])


def kernel(input_ids, attention_mask, emb_table, W, b):
    del attention_mask
    p = _project(emb_table, W)
    return _gather_mean(input_ids.astype(jnp.int32), p, b)


# speedup vs baseline: 17.8315x; 1.0587x over previous
import functools

import jax
import jax.numpy as jnp
from jax import lax
from jax.experimental import pallas as pl
from jax.experimental.pallas import tpu as pltpu
from jax.experimental.pallas import tpu_sc as plsc

VOCAB = 30522
EMBED_DIM = 128
SEQ_LEN = 512
BATCH = 1024
FC_OUT = 64

HALF = 15360
PVIEW = 2 * HALF

NUM_CORES = 2
NUM_SUBCORES = 16
NW = NUM_CORES * NUM_SUBCORES
BPW = BATCH // NW
CHUNK = 128
CPB = SEQ_LEN // CHUNK
NCH = BPW * CPB
LANES = 16
NVR = FC_OUT // LANES

ROW_BLK = 512


def _project_kernel(lo_ref, hi_ref, w_ref, p2_ref):
    p2_ref[:, 0:FC_OUT] = jnp.dot(lo_ref[...], w_ref[...],
                                  preferred_element_type=jnp.float32)
    p2_ref[:, FC_OUT:2 * FC_OUT] = jnp.dot(hi_ref[...], w_ref[...],
                                           preferred_element_type=jnp.float32)


def _project(table, w):
    grid = (HALF // ROW_BLK,)
    return pl.pallas_call(
        _project_kernel,
        grid=grid,
        in_specs=[
            pl.BlockSpec((ROW_BLK, EMBED_DIM), lambda i: (i, 0)),
            pl.BlockSpec((ROW_BLK, EMBED_DIM),
                         lambda i: (i + HALF // ROW_BLK, 0)),
            pl.BlockSpec((EMBED_DIM, FC_OUT), lambda i: (0, 0)),
        ],
        out_specs=pl.BlockSpec((ROW_BLK, 2 * FC_OUT), lambda i: (i, 0)),
        out_shape=jax.ShapeDtypeStruct((HALF, 2 * FC_OUT), jnp.float32),
    )(table, table, w)


_MESH = plsc.VectorSubcoreMesh(core_axis_name="c", subcore_axis_name="s",
                               num_cores=NUM_CORES, num_subcores=NUM_SUBCORES)


@functools.partial(
    pl.kernel,
    out_type=jax.ShapeDtypeStruct((BATCH, FC_OUT), jnp.float32),
    mesh=_MESH,
    scratch_types=[
        pltpu.VMEM((NCH, CHUNK), jnp.int32),
        pltpu.VMEM((CPB, CHUNK, FC_OUT), jnp.float32),
        pltpu.VMEM((BPW, FC_OUT), jnp.float32),
        pltpu.VMEM((FC_OUT,), jnp.float32),
        pltpu.SemaphoreType.DMA,
        pltpu.SemaphoreType.DMA,
        pltpu.SemaphoreType.DMA,
        pltpu.SemaphoreType.DMA,
    ],
    compiler_params=pltpu.CompilerParams(use_tc_tiling_on_sc=False,
                                         needs_layout_passes=False),
)
def _gather_mean(ids_hbm, p_hbm, b_hbm, out_hbm,
                 idx_v, rows_v, out_v, bias_v, s0, s1, s2, s3):
    sems = (s0, s1, s2, s3)
    wid = lax.axis_index("s") * NUM_CORES + lax.axis_index("c")

    pltpu.sync_copy(b_hbm, bias_v)
    pltpu.sync_copy(ids_hbm.at[pl.ds(wid * NCH, NCH)], idx_v)

    def _copy(j, c):
        return pltpu.make_async_copy(
            p_hbm.at[idx_v.at[j]], rows_v.at[c], sems[c])

    for c in range(CPB):
        _copy(c, c).start()

    inv = jnp.float32(1.0 / SEQ_LEN)

    def batch_body(bi, _):
        accs = [jnp.zeros((LANES,), jnp.float32) for _ in range(NVR)]
        for c in range(CPB):
            j = bi * CPB + c
            _copy(j, c).wait()

            def row_body(r, a):
                return tuple(
                    a[k] + rows_v[c, r, pl.ds(k * LANES, LANES)]
                    for k in range(NVR))

            accs = list(lax.fori_loop(0, CHUNK, row_body, tuple(accs),
                                      unroll=8))

            @pl.when(bi + 1 < BPW)
            def _():
                _copy(j + CPB, c).start()

        for k in range(NVR):
            v = accs[k] * inv + bias_v[pl.ds(k * LANES, LANES)]
            out_v[bi, pl.ds(k * LANES, LANES)] = jnp.maximum(v, 0.0)
        return 0

    lax.fori_loop(0, BPW, batch_body, 0)
    pltpu.sync_copy(out_v, out_hbm.at[pl.ds(wid * BPW, BPW)])


def kernel(input_ids, attention_mask, emb_table, W, b):
    del attention_mask
    p2 = _project(emb_table, W)
    pview = p2.reshape(PVIEW, FC_OUT)
    ids = input_ids.astype(jnp.int32)
    idsv = jnp.where(ids < HALF, ids * 2, ids * 2 - (2 * HALF - 1))
    return _gather_mean(idsv.reshape(-1, CHUNK), pview, b)


# speedup vs baseline: 19.3828x; 1.0870x over previous
import functools

import jax
import jax.numpy as jnp
from jax import lax
from jax.experimental import pallas as pl
from jax.experimental.pallas import tpu as pltpu
from jax.experimental.pallas import tpu_sc as plsc

VOCAB = 30522
EMBED_DIM = 128
SEQ_LEN = 512
BATCH = 1024
FC_OUT = 64

HALF = 15360
PVIEW = 2 * HALF

NUM_CORES = 2
NUM_SUBCORES = 16
NW = NUM_CORES * NUM_SUBCORES
BPW = BATCH // NW
CHUNK = 128
CPB = SEQ_LEN // CHUNK
NCH = BPW * CPB
LANES = 16
NVR = FC_OUT // LANES

ROW_BLK = 1024


def _project_kernel(lo_ref, hi_ref, w_ref, p2_ref):
    p2_ref[:, 0:FC_OUT] = jnp.dot(lo_ref[...], w_ref[...],
                                  preferred_element_type=jnp.float32)
    p2_ref[:, FC_OUT:2 * FC_OUT] = jnp.dot(hi_ref[...], w_ref[...],
                                           preferred_element_type=jnp.float32)


def _project(table, w):
    grid = (HALF // ROW_BLK,)
    return pl.pallas_call(
        _project_kernel,
        grid=grid,
        in_specs=[
            pl.BlockSpec((ROW_BLK, EMBED_DIM), lambda i: (i, 0)),
            pl.BlockSpec((ROW_BLK, EMBED_DIM),
                         lambda i: (i + HALF // ROW_BLK, 0)),
            pl.BlockSpec((EMBED_DIM, FC_OUT), lambda i: (0, 0)),
        ],
        out_specs=pl.BlockSpec((ROW_BLK, 2 * FC_OUT), lambda i: (i, 0)),
        out_shape=jax.ShapeDtypeStruct((HALF, 2 * FC_OUT), jnp.float32),
    )(table, table, w)


_MESH = plsc.VectorSubcoreMesh(core_axis_name="c", subcore_axis_name="s",
                               num_cores=NUM_CORES, num_subcores=NUM_SUBCORES)


@functools.partial(
    pl.kernel,
    out_type=jax.ShapeDtypeStruct((BATCH, FC_OUT), jnp.float32),
    mesh=_MESH,
    scratch_types=[
        pltpu.VMEM((NCH, CHUNK), jnp.int32),
        pltpu.VMEM((CPB, CHUNK, FC_OUT), jnp.float32),
        pltpu.VMEM((BPW, FC_OUT), jnp.float32),
        pltpu.VMEM((FC_OUT,), jnp.float32),
        pltpu.SemaphoreType.DMA,
        pltpu.SemaphoreType.DMA,
        pltpu.SemaphoreType.DMA,
        pltpu.SemaphoreType.DMA,
    ],
    compiler_params=pltpu.CompilerParams(use_tc_tiling_on_sc=False,
                                         needs_layout_passes=False),
)
def _gather_mean(ids_hbm, p_hbm, b_hbm, out_hbm,
                 idx_v, rows_v, out_v, bias_v, s0, s1, s2, s3):
    sems = (s0, s1, s2, s3)
    wid = lax.axis_index("s") * NUM_CORES + lax.axis_index("c")

    pltpu.sync_copy(b_hbm, bias_v)
    pltpu.sync_copy(ids_hbm.at[pl.ds(wid * NCH, NCH)], idx_v)

    def _copy(j, c):
        return pltpu.make_async_copy(
            p_hbm.at[idx_v.at[j]], rows_v.at[c], sems[c])

    for c in range(CPB):
        _copy(c, c).start()

    inv = jnp.float32(1.0 / SEQ_LEN)

    def batch_body(bi, _):
        accs = [jnp.zeros((LANES,), jnp.float32) for _ in range(NVR)]
        for c in range(CPB):
            j = bi * CPB + c
            _copy(j, c).wait()

            def row_body(r, a):
                return tuple(
                    a[k] + rows_v[c, r, pl.ds(k * LANES, LANES)]
                    for k in range(NVR))

            accs = list(lax.fori_loop(0, CHUNK, row_body, tuple(accs),
                                      unroll=8))

            @pl.when(bi + 1 < BPW)
            def _():
                _copy(j + CPB, c).start()

        for k in range(NVR):
            v = accs[k] * inv + bias_v[pl.ds(k * LANES, LANES)]
            out_v[bi, pl.ds(k * LANES, LANES)] = jnp.maximum(v, 0.0)
        return 0

    lax.fori_loop(0, BPW, batch_body, 0)
    pltpu.sync_copy(out_v, out_hbm.at[pl.ds(wid * BPW, BPW)])


def kernel(input_ids, attention_mask, emb_table, W, b):
    del attention_mask
    p2 = _project(emb_table, W)
    pview = p2.reshape(PVIEW, FC_OUT)
    ids = input_ids.astype(jnp.int32)
    idsv = jnp.where(ids < HALF, ids * 2, ids * 2 - (2 * HALF - 1))
    return _gather_mean(idsv.reshape(-1, CHUNK), pview, b)


# speedup vs baseline: 20.5343x; 1.0594x over previous
import functools

import jax
import jax.numpy as jnp
from jax import lax
from jax.experimental import pallas as pl
from jax.experimental.pallas import tpu as pltpu
from jax.experimental.pallas import tpu_sc as plsc

VOCAB = 30522
EMBED_DIM = 128
SEQ_LEN = 512
BATCH = 1024
FC_OUT = 64

HALF = 15360
PVIEW = 2 * HALF

NUM_CORES = 2
NUM_SUBCORES = 16
NW = NUM_CORES * NUM_SUBCORES
BPW = BATCH // NW
CHUNK = 128
CPB = SEQ_LEN // CHUNK
NCH = BPW * CPB
LANES = 16
NVR = FC_OUT // LANES

ROW_BLK = 2560


def _project_kernel(lo_ref, hi_ref, w_ref, p2_ref):
    p2_ref[:, 0:FC_OUT] = jnp.dot(lo_ref[...], w_ref[...],
                                  preferred_element_type=jnp.float32)
    p2_ref[:, FC_OUT:2 * FC_OUT] = jnp.dot(hi_ref[...], w_ref[...],
                                           preferred_element_type=jnp.float32)


def _project(table, w):
    grid = (HALF // ROW_BLK,)
    return pl.pallas_call(
        _project_kernel,
        grid=grid,
        in_specs=[
            pl.BlockSpec((ROW_BLK, EMBED_DIM), lambda i: (i, 0)),
            pl.BlockSpec((ROW_BLK, EMBED_DIM),
                         lambda i: (i + HALF // ROW_BLK, 0)),
            pl.BlockSpec((EMBED_DIM, FC_OUT), lambda i: (0, 0)),
        ],
        out_specs=pl.BlockSpec((ROW_BLK, 2 * FC_OUT), lambda i: (i, 0)),
        out_shape=jax.ShapeDtypeStruct((HALF, 2 * FC_OUT), jnp.float32),
    )(table, table, w)


_MESH = plsc.VectorSubcoreMesh(core_axis_name="c", subcore_axis_name="s",
                               num_cores=NUM_CORES, num_subcores=NUM_SUBCORES)


@functools.partial(
    pl.kernel,
    out_type=jax.ShapeDtypeStruct((BATCH, FC_OUT), jnp.float32),
    mesh=_MESH,
    scratch_types=[
        pltpu.VMEM((NCH, CHUNK), jnp.int32),
        pltpu.VMEM((CPB, CHUNK, FC_OUT), jnp.float32),
        pltpu.VMEM((BPW, FC_OUT), jnp.float32),
        pltpu.VMEM((FC_OUT,), jnp.float32),
        pltpu.SemaphoreType.DMA,
        pltpu.SemaphoreType.DMA,
        pltpu.SemaphoreType.DMA,
        pltpu.SemaphoreType.DMA,
    ],
    compiler_params=pltpu.CompilerParams(use_tc_tiling_on_sc=False,
                                         needs_layout_passes=False),
)
def _gather_mean(ids_hbm, p_hbm, b_hbm, out_hbm,
                 idx_v, rows_v, out_v, bias_v, s0, s1, s2, s3):
    sems = (s0, s1, s2, s3)
    wid = lax.axis_index("s") * NUM_CORES + lax.axis_index("c")

    pltpu.sync_copy(b_hbm, bias_v)
    pltpu.sync_copy(ids_hbm.at[pl.ds(wid * NCH, NCH)], idx_v)

    def _copy(j, c):
        return pltpu.make_async_copy(
            p_hbm.at[idx_v.at[j]], rows_v.at[c], sems[c])

    for c in range(CPB):
        _copy(c, c).start()

    inv = jnp.float32(1.0 / SEQ_LEN)

    def batch_body(bi, _):
        accs = [jnp.zeros((LANES,), jnp.float32) for _ in range(NVR)]
        for c in range(CPB):
            j = bi * CPB + c
            _copy(j, c).wait()

            def row_body(r, a):
                return tuple(
                    a[k] + rows_v[c, r, pl.ds(k * LANES, LANES)]
                    for k in range(NVR))

            accs = list(lax.fori_loop(0, CHUNK, row_body, tuple(accs),
                                      unroll=8))

            @pl.when(bi + 1 < BPW)
            def _():
                _copy(j + CPB, c).start()

        for k in range(NVR):
            v = accs[k] * inv + bias_v[pl.ds(k * LANES, LANES)]
            out_v[bi, pl.ds(k * LANES, LANES)] = jnp.maximum(v, 0.0)
        return 0

    lax.fori_loop(0, BPW, batch_body, 0)
    pltpu.sync_copy(out_v, out_hbm.at[pl.ds(wid * BPW, BPW)])


def kernel(input_ids, attention_mask, emb_table, W, b):
    del attention_mask
    p2 = _project(emb_table, W)
    pview = p2.reshape(PVIEW, FC_OUT)
    ids = input_ids.astype(jnp.int32)
    idsv = jnp.where(ids < HALF, ids * 2, ids * 2 - (2 * HALF - 1))
    return _gather_mean(idsv.reshape(-1, CHUNK), pview, b)


# speedup vs baseline: 20.6651x; 1.0064x over previous
import functools

import jax
import jax.numpy as jnp
from jax import lax
from jax.experimental import pallas as pl
from jax.experimental.pallas import tpu as pltpu
from jax.experimental.pallas import tpu_sc as plsc

VOCAB = 30522
EMBED_DIM = 128
SEQ_LEN = 512
BATCH = 1024
FC_OUT = 64

HALF = 15360
PVIEW = 2 * HALF

NUM_CORES = 2
NUM_SUBCORES = 16
NW = NUM_CORES * NUM_SUBCORES
BPW = BATCH // NW
CHUNK = 128
CPB = SEQ_LEN // CHUNK
NCH = BPW * CPB
LANES = 16
NVR = FC_OUT // LANES

ROW_BLK = 5120


def _project_kernel(lo_ref, hi_ref, w_ref, p2_ref):
    p2_ref[:, 0:FC_OUT] = jnp.dot(lo_ref[...], w_ref[...],
                                  preferred_element_type=jnp.float32)
    p2_ref[:, FC_OUT:2 * FC_OUT] = jnp.dot(hi_ref[...], w_ref[...],
                                           preferred_element_type=jnp.float32)


def _project(table, w):
    grid = (HALF // ROW_BLK,)
    return pl.pallas_call(
        _project_kernel,
        grid=grid,
        in_specs=[
            pl.BlockSpec((ROW_BLK, EMBED_DIM), lambda i: (i, 0)),
            pl.BlockSpec((ROW_BLK, EMBED_DIM),
                         lambda i: (i + HALF // ROW_BLK, 0)),
            pl.BlockSpec((EMBED_DIM, FC_OUT), lambda i: (0, 0)),
        ],
        out_specs=pl.BlockSpec((ROW_BLK, 2 * FC_OUT), lambda i: (i, 0)),
        out_shape=jax.ShapeDtypeStruct((HALF, 2 * FC_OUT), jnp.float32),
    )(table, table, w)


_MESH = plsc.VectorSubcoreMesh(core_axis_name="c", subcore_axis_name="s",
                               num_cores=NUM_CORES, num_subcores=NUM_SUBCORES)


@functools.partial(
    pl.kernel,
    out_type=jax.ShapeDtypeStruct((BATCH, FC_OUT), jnp.float32),
    mesh=_MESH,
    scratch_types=[
        pltpu.VMEM((NCH, CHUNK), jnp.int32),
        pltpu.VMEM((CPB, CHUNK, FC_OUT), jnp.float32),
        pltpu.VMEM((BPW, FC_OUT), jnp.float32),
        pltpu.VMEM((FC_OUT,), jnp.float32),
        pltpu.SemaphoreType.DMA,
        pltpu.SemaphoreType.DMA,
        pltpu.SemaphoreType.DMA,
        pltpu.SemaphoreType.DMA,
    ],
    compiler_params=pltpu.CompilerParams(use_tc_tiling_on_sc=False,
                                         needs_layout_passes=False),
)
def _gather_mean(ids_hbm, p_hbm, b_hbm, out_hbm,
                 idx_v, rows_v, out_v, bias_v, s0, s1, s2, s3):
    sems = (s0, s1, s2, s3)
    wid = lax.axis_index("s") * NUM_CORES + lax.axis_index("c")

    pltpu.sync_copy(b_hbm, bias_v)
    pltpu.sync_copy(ids_hbm.at[pl.ds(wid * NCH, NCH)], idx_v)

    def _copy(j, c):
        return pltpu.make_async_copy(
            p_hbm.at[idx_v.at[j]], rows_v.at[c], sems[c])

    for c in range(CPB):
        _copy(c, c).start()

    inv = jnp.float32(1.0 / SEQ_LEN)

    def batch_body(bi, _):
        accs = [jnp.zeros((LANES,), jnp.float32) for _ in range(NVR)]
        for c in range(CPB):
            j = bi * CPB + c
            _copy(j, c).wait()

            def row_body(r, a):
                return tuple(
                    a[k] + rows_v[c, r, pl.ds(k * LANES, LANES)]
                    for k in range(NVR))

            accs = list(lax.fori_loop(0, CHUNK, row_body, tuple(accs),
                                      unroll=8))

            @pl.when(bi + 1 < BPW)
            def _():
                _copy(j + CPB, c).start()

        for k in range(NVR):
            v = accs[k] * inv + bias_v[pl.ds(k * LANES, LANES)]
            out_v[bi, pl.ds(k * LANES, LANES)] = jnp.maximum(v, 0.0)
        return 0

    lax.fori_loop(0, BPW, batch_body, 0)
    pltpu.sync_copy(out_v, out_hbm.at[pl.ds(wid * BPW, BPW)])


def kernel(input_ids, attention_mask, emb_table, W, b):
    del attention_mask
    p2 = _project(emb_table, W)
    pview = p2.reshape(PVIEW, FC_OUT)
    ids = input_ids.astype(jnp.int32)
    idsv = jnp.where(ids < HALF, ids * 2, ids * 2 - (2 * HALF - 1))
    return _gather_mean(idsv.reshape(-1, CHUNK), pview, b)


# speedup vs baseline: 20.7074x; 1.0020x over previous
import functools

import jax
import jax.numpy as jnp
from jax import lax
from jax.experimental import pallas as pl
from jax.experimental.pallas import tpu as pltpu
from jax.experimental.pallas import tpu_sc as plsc

VOCAB = 30522
EMBED_DIM = 128
SEQ_LEN = 512
BATCH = 1024
FC_OUT = 64

HALF = 15360
PVIEW = 2 * HALF

NUM_CORES = 2
NUM_SUBCORES = 16
NW = NUM_CORES * NUM_SUBCORES
BPW = BATCH // NW
CHUNK = 128
CPB = SEQ_LEN // CHUNK
NCH = BPW * CPB
LANES = 16
NVR = FC_OUT // LANES

ROW_BLK = 5120


def _project_kernel(lo_ref, hi_ref, w_ref, p2_ref):
    p2_ref[:, 0:FC_OUT] = jnp.dot(lo_ref[...], w_ref[...],
                                  preferred_element_type=jnp.float32)
    p2_ref[:, FC_OUT:2 * FC_OUT] = jnp.dot(hi_ref[...], w_ref[...],
                                           preferred_element_type=jnp.float32)


def _project(table, w):
    grid = (HALF // ROW_BLK,)
    return pl.pallas_call(
        _project_kernel,
        grid=grid,
        in_specs=[
            pl.BlockSpec((ROW_BLK, EMBED_DIM), lambda i: (i, 0)),
            pl.BlockSpec((ROW_BLK, EMBED_DIM),
                         lambda i: (i + HALF // ROW_BLK, 0)),
            pl.BlockSpec((EMBED_DIM, FC_OUT), lambda i: (0, 0)),
        ],
        out_specs=pl.BlockSpec((ROW_BLK, 2 * FC_OUT), lambda i: (i, 0)),
        out_shape=jax.ShapeDtypeStruct((HALF, 2 * FC_OUT), jnp.float32),
    )(table, table, w)


_MESH = plsc.VectorSubcoreMesh(core_axis_name="c", subcore_axis_name="s",
                               num_cores=NUM_CORES, num_subcores=NUM_SUBCORES)


@functools.partial(
    pl.kernel,
    out_type=jax.ShapeDtypeStruct((BATCH, FC_OUT), jnp.float32),
    mesh=_MESH,
    scratch_types=[
        pltpu.VMEM((NCH, CHUNK), jnp.int32),
        pltpu.VMEM((2 * CPB, CHUNK, FC_OUT), jnp.float32),
        pltpu.VMEM((BPW, FC_OUT), jnp.float32),
        pltpu.VMEM((FC_OUT,), jnp.float32),
        pltpu.SemaphoreType.DMA,
        pltpu.SemaphoreType.DMA,
        pltpu.SemaphoreType.DMA,
        pltpu.SemaphoreType.DMA,
        pltpu.SemaphoreType.DMA,
        pltpu.SemaphoreType.DMA,
        pltpu.SemaphoreType.DMA,
        pltpu.SemaphoreType.DMA,
    ],
    compiler_params=pltpu.CompilerParams(use_tc_tiling_on_sc=False,
                                         needs_layout_passes=False),
)
def _gather_mean(ids_hbm, p_hbm, b_hbm, out_hbm,
                 idx_v, rows_v, out_v, bias_v,
                 s0, s1, s2, s3, s4, s5, s6, s7):
    sems = (s0, s1, s2, s3, s4, s5, s6, s7)
    nbuf = 2 * CPB
    wid = lax.axis_index("s") * NUM_CORES + lax.axis_index("c")

    pltpu.sync_copy(b_hbm, bias_v)
    pltpu.sync_copy(ids_hbm.at[pl.ds(wid * NCH, NCH)], idx_v)

    def _copy(j, b):
        return pltpu.make_async_copy(
            p_hbm.at[idx_v.at[j]], rows_v.at[b], sems[b])

    for b in range(nbuf):
        _copy(b, b).start()

    inv = jnp.float32(1.0 / SEQ_LEN)

    def pair_body(bi2, _):
        for half in range(2):
            bi = bi2 * 2 + half
            accs = [jnp.zeros((LANES,), jnp.float32) for _ in range(NVR)]
            for c in range(CPB):
                b = half * CPB + c
                j = bi * CPB + c
                _copy(j, b).wait()

                def row_body(r, a):
                    return tuple(
                        a[k] + rows_v[b, r, pl.ds(k * LANES, LANES)]
                        for k in range(NVR))

                accs = list(lax.fori_loop(0, CHUNK, row_body, tuple(accs),
                                          unroll=8))

                @pl.when(bi2 + 1 < BPW // 2)
                def _():
                    _copy(j + nbuf, b).start()

            for k in range(NVR):
                v = accs[k] * inv + bias_v[pl.ds(k * LANES, LANES)]
                out_v[bi, pl.ds(k * LANES, LANES)] = jnp.maximum(v, 0.0)
        return 0

    lax.fori_loop(0, BPW // 2, pair_body, 0)
    pltpu.sync_copy(out_v, out_hbm.at[pl.ds(wid * BPW, BPW)])


def kernel(input_ids, attention_mask, emb_table, W, b):
    del attention_mask
    p2 = _project(emb_table, W)
    pview = p2.reshape(PVIEW, FC_OUT)
    ids = input_ids.astype(jnp.int32)
    idsv = jnp.where(ids < HALF, ids * 2, ids * 2 - (2 * HALF - 1))
    return _gather_mean(idsv.reshape(-1, CHUNK), pview, b)
